# baseline (speedup 1.0000x reference)
import jax
import jax.numpy as jnp
from jax.experimental import pallas as pl
from jax.experimental.pallas import tpu as pltpu

_B, _N, _F = 16, 4096, 512
_E = 64


def _unpool_kernel(a_ref, o_ref, lo_ref, hi_ref, tail_ref,
                   body_sem, lo_sem, hi_sem, tail_sem):
    lo_cp = pltpu.make_async_copy(a_ref.at[:, 0:_E, :], lo_ref, lo_sem)
    hi_cp = pltpu.make_async_copy(a_ref.at[:, 2048:2048 + _E, :], hi_ref, hi_sem)
    lo_cp.start()
    hi_cp.start()

    for b in range(_B):
        pltpu.make_async_copy(
            a_ref.at[b], o_ref.at[b, 0:_N, :], body_sem).start()

    lo_cp.wait()
    hi_cp.wait()
    tail_ref[...] = 0.5 * (lo_ref[...] + hi_ref[...])
    tail_cp = pltpu.make_async_copy(tail_ref, o_ref.at[:, _N:_N + _E, :], tail_sem)
    tail_cp.start()

    for b in range(_B):
        pltpu.make_async_copy(
            a_ref.at[b], o_ref.at[b, 0:_N, :], body_sem).wait()
    tail_cp.wait()


def kernel(inputs):
    return pl.pallas_call(
        _unpool_kernel,
        in_specs=[pl.BlockSpec(memory_space=pl.ANY)],
        out_specs=pl.BlockSpec(memory_space=pl.ANY),
        out_shape=jax.ShapeDtypeStruct((_B, _N + _E, _F), inputs.dtype),
        scratch_shapes=[
            pltpu.VMEM((_B, _E, _F), inputs.dtype),
            pltpu.VMEM((_B, _E, _F), inputs.dtype),
            pltpu.VMEM((_B, _E, _F), inputs.dtype),
            pltpu.SemaphoreType.DMA,
            pltpu.SemaphoreType.DMA,
            pltpu.SemaphoreType.DMA,
            pltpu.SemaphoreType.DMA,
        ],
    )(inputs)


# speedup vs baseline: 48.3948x; 48.3948x over previous
import jax
import jax.numpy as jnp
from jax.experimental import pallas as pl
from jax.experimental.pallas import tpu as pltpu

_B, _N, _F = 16, 4096, 512
_E = 64
_NBUF = 4


def _unpool_kernel(a_ref, o_ref, b0, b1, b2, b3, lo_ref, hi_ref, tail_ref,
                   in_sems, out_sems, lo_sem, hi_sem, tail_sem):
    bufs = (b0, b1, b2, b3)

    def in_cp(b):
        return pltpu.make_async_copy(a_ref.at[b], bufs[b % _NBUF],
                                     in_sems.at[b % _NBUF])

    def out_cp(b):
        return pltpu.make_async_copy(bufs[b % _NBUF], o_ref.at[b, 0:_N, :],
                                     out_sems.at[b % _NBUF])

    lo_cp = pltpu.make_async_copy(a_ref.at[:, 0:_E, :], lo_ref, lo_sem)
    hi_cp = pltpu.make_async_copy(a_ref.at[:, 2048:2048 + _E, :], hi_ref, hi_sem)
    lo_cp.start()
    hi_cp.start()
    in_cp(0).start()
    in_cp(1).start()

    lo_cp.wait()
    hi_cp.wait()
    tail_ref[...] = 0.5 * (lo_ref[...] + hi_ref[...])
    tail_cp = pltpu.make_async_copy(tail_ref, o_ref.at[:, _N:_N + _E, :], tail_sem)
    tail_cp.start()

    for b in range(_B):
        if b >= 2:
            out_cp(b - 2).wait()
        if b + 2 < _B:
            in_cp(b + 2).start()
        in_cp(b).wait()
        out_cp(b).start()
    out_cp(_B - 2).wait()
    out_cp(_B - 1).wait()
    tail_cp.wait()


def kernel(inputs):
    return pl.pallas_call(
        _unpool_kernel,
        in_specs=[pl.BlockSpec(memory_space=pl.ANY)],
        out_specs=pl.BlockSpec(memory_space=pl.ANY),
        out_shape=jax.ShapeDtypeStruct((_B, _N + _E, _F), inputs.dtype),
        scratch_shapes=[
            pltpu.VMEM((_N, _F), inputs.dtype),
            pltpu.VMEM((_N, _F), inputs.dtype),
            pltpu.VMEM((_N, _F), inputs.dtype),
            pltpu.VMEM((_N, _F), inputs.dtype),
            pltpu.VMEM((_B, _E, _F), inputs.dtype),
            pltpu.VMEM((_B, _E, _F), inputs.dtype),
            pltpu.VMEM((_B, _E, _F), inputs.dtype),
            pltpu.SemaphoreType.DMA((_NBUF,)),
            pltpu.SemaphoreType.DMA((_NBUF,)),
            pltpu.SemaphoreType.DMA,
            pltpu.SemaphoreType.DMA,
            pltpu.SemaphoreType.DMA,
        ],
    )(inputs)
